# baseline (speedup 1.0000x reference)
import functools

import jax
import jax.numpy as jnp
from jax import lax
from jax.experimental import pallas as pl
from jax.experimental.pallas import tpu as pltpu
from jax.experimental.pallas import tpu_sc as plsc

BATCH = 16384
D_MLP = 256
D_GMF = 64

_NC = 2
_NS = 16
_NW = _NC * _NS
_BPW = BATCH // _NW
_CHUNK = 32
_NCHUNK = _BPW // _CHUNK

_sc_mesh = plsc.VectorSubcoreMesh(core_axis_name="c", subcore_axis_name="s")


@functools.partial(
    pl.kernel,
    mesh=_sc_mesh,
    out_type=[
        jax.ShapeDtypeStruct((BATCH, D_MLP), jnp.float32),
        jax.ShapeDtypeStruct((BATCH, D_MLP), jnp.float32),
        jax.ShapeDtypeStruct((BATCH, D_GMF), jnp.float32),
    ],
    scratch_types=[
        pltpu.VMEM((_BPW,), jnp.int32),
        pltpu.VMEM((_BPW,), jnp.int32),
        pltpu.VMEM((_BPW,), jnp.int32),
        pltpu.VMEM((_BPW,), jnp.int32),
        pltpu.VMEM((2, _CHUNK, D_MLP), jnp.float32),
        pltpu.VMEM((2, _CHUNK, D_MLP), jnp.float32),
        pltpu.VMEM((2, _CHUNK, 4 * D_GMF), jnp.float32),
        pltpu.VMEM((2, _CHUNK, 4 * D_GMF), jnp.float32),
        pltpu.VMEM((_CHUNK, D_GMF), jnp.float32),
        pltpu.SemaphoreType.DMA,
        pltpu.SemaphoreType.DMA,
    ],
    compiler_params=pltpu.CompilerParams(needs_layout_passes=False),
)
def _sc_gather(users_hbm, items_hbm, uemb_hbm, iemb_hbm, pair_hbm,
               out_u, out_i, out_g,
               uidx_v, iidx_v, uhalf_v, ihalf_v, urows_v, irows_v,
               ucat_v, icat_v, g_v, sem0, sem1):
    wid = lax.axis_index("s") * _NC + lax.axis_index("c")
    base = wid * _BPW
    sems = (sem0, sem1)

    pltpu.sync_copy(users_hbm.at[pl.ds(base, _BPW)], uidx_v)
    pltpu.sync_copy(items_hbm.at[pl.ds(base, _BPW)], iidx_v)

    def half_body(s, hc):
        sl = pl.ds(s * 16, 16)
        uhalf_v[sl] = lax.shift_right_logical(uidx_v[sl], 1)
        ihalf_v[sl] = lax.shift_right_logical(iidx_v[sl], 1)
        return hc

    lax.fori_loop(0, _BPW // 16, half_body, 0)

    def fire(k):
        p = k % 2
        uix = uidx_v.at[pl.ds(k * _CHUNK, _CHUNK)]
        iix = iidx_v.at[pl.ds(k * _CHUNK, _CHUNK)]
        uhx = uhalf_v.at[pl.ds(k * _CHUNK, _CHUNK)]
        ihx = ihalf_v.at[pl.ds(k * _CHUNK, _CHUNK)]
        return (
            pltpu.async_copy(uemb_hbm.at[uix], urows_v.at[p], sems[p]),
            pltpu.async_copy(iemb_hbm.at[iix], irows_v.at[p], sems[p]),
            pltpu.async_copy(pair_hbm.at[uhx], ucat_v.at[p], sems[p]),
            pltpu.async_copy(pair_hbm.at[ihx], icat_v.at[p], sems[p]),
        )

    lanes = lax.broadcasted_iota(jnp.int32, (16,), 0)
    inflight = fire(0)
    for k in range(_NCHUNK):
        nxt = fire(k + 1) if k + 1 < _NCHUNK else None
        for c in inflight:
            c.wait()
        p = k % 2
        off = base + k * _CHUNK

        def mul_body(r, mc):
            q = k * _CHUNK + r
            grp = pl.ds((q // 16) * 16, 16)
            lane = q % 16
            dz = 1 - jnp.minimum(jnp.abs(lanes - lane), 1)
            uval = jnp.sum(uidx_v[grp] * dz)
            ival = jnp.sum(iidx_v[grp] * dz)
            pu = lax.convert_element_type(lax.bitwise_and(uval, 1),
                                          jnp.float32)
            pi = lax.convert_element_type(lax.bitwise_and(ival, 1),
                                          jnp.float32)
            puv = jnp.full((16,), pu)
            qu = 1.0 - puv
            piv = jnp.full((16,), pi)
            qi = 1.0 - piv
            for c in range(D_GMF // 16):
                u_lo = pl.ds(c * 16, 16)
                u_hi = pl.ds(D_GMF + c * 16, 16)
                i_lo = pl.ds(2 * D_GMF + c * 16, 16)
                i_hi = pl.ds(3 * D_GMF + c * 16, 16)
                uu = ucat_v[p, r, u_lo] * qu + ucat_v[p, r, u_hi] * puv
                ii = icat_v[p, r, i_lo] * qi + icat_v[p, r, i_hi] * piv
                g_v[r, u_lo] = uu * ii
            return mc

        lax.fori_loop(0, _CHUNK, mul_body, 0)

        pltpu.sync_copy(urows_v.at[p], out_u.at[pl.ds(off, _CHUNK)])
        pltpu.sync_copy(irows_v.at[p], out_i.at[pl.ds(off, _CHUNK)])
        pltpu.sync_copy(g_v, out_g.at[pl.ds(off, _CHUNK)])
        inflight = nxt


def _dense_body(u_ref, i_ref, g_ref, w0a_ref, w0b_ref, b0_ref, w1_ref,
                b1_ref, w2_ref, b2_ref, pwa_ref, pwb_ref, pb_ref, o_ref):
    h = jnp.dot(u_ref[...], w0a_ref[...], preferred_element_type=jnp.float32)
    h += jnp.dot(i_ref[...], w0b_ref[...], preferred_element_type=jnp.float32)
    h = jnp.maximum(h + b0_ref[...], 0.0)
    h = jnp.dot(h, w1_ref[...], preferred_element_type=jnp.float32)
    h = jnp.maximum(h + b1_ref[...], 0.0)
    h = jnp.dot(h, w2_ref[...], preferred_element_type=jnp.float32)
    h = jnp.maximum(h + b2_ref[...], 0.0)
    logit = jnp.dot(h, pwa_ref[...], preferred_element_type=jnp.float32)
    logit += jnp.dot(g_ref[...], pwb_ref[...], preferred_element_type=jnp.float32)
    logit += pb_ref[0, 0]
    o_ref[...] = 1.0 / (1.0 + jnp.exp(-logit))


def _dense(u_rows, i_rows, g, w0a, w0b, b0, w1, b1, w2, b2, pwa, pwb, pb,
           block_m=2048):
    grid = (BATCH // block_m,)
    full = lambda m: (0, 0)
    return pl.pallas_call(
        _dense_body,
        grid=grid,
        in_specs=[
            pl.BlockSpec((block_m, D_MLP), lambda m: (m, 0)),
            pl.BlockSpec((block_m, D_MLP), lambda m: (m, 0)),
            pl.BlockSpec((block_m, D_GMF), lambda m: (m, 0)),
            pl.BlockSpec((D_MLP, 256), full),
            pl.BlockSpec((D_MLP, 256), full),
            pl.BlockSpec((1, 256), full),
            pl.BlockSpec((256, 128), full),
            pl.BlockSpec((1, 128), full),
            pl.BlockSpec((128, 64), full),
            pl.BlockSpec((1, 64), full),
            pl.BlockSpec((64, 1), full),
            pl.BlockSpec((64, 1), full),
            pl.BlockSpec((1, 1), full),
        ],
        out_specs=pl.BlockSpec((block_m, 1), lambda m: (m, 0)),
        out_shape=jax.ShapeDtypeStruct((BATCH, 1), jnp.float32),
        compiler_params=pltpu.CompilerParams(
            dimension_semantics=("arbitrary",),
        ),
    )(u_rows, i_rows, g, w0a, w0b, b0, w1, b1, w2, b2, pwa, pwb, pb)


def kernel(users, items, user_emb_mlp, item_emb_mlp, user_emb_gmf,
           item_emb_gmf, mlp_w0, mlp_b0, mlp_w1, mlp_b1, mlp_w2, mlp_b2,
           pred_w, pred_b):
    users = users.astype(jnp.int32)
    items = items.astype(jnp.int32)

    pair = jnp.concatenate(
        [user_emb_gmf.reshape(-1, 2 * D_GMF),
         item_emb_gmf.reshape(-1, 2 * D_GMF)], axis=1)
    u_rows, i_rows, g = _sc_gather(users, items, user_emb_mlp, item_emb_mlp,
                                   pair)

    w0a = mlp_w0[:D_MLP]
    w0b = mlp_w0[D_MLP:]
    pwa = pred_w[:D_GMF]
    pwb = pred_w[D_GMF:]
    out = _dense(u_rows, i_rows, g, w0a, w0b, mlp_b0.reshape(1, -1),
                 mlp_w1, mlp_b1.reshape(1, -1), mlp_w2,
                 mlp_b2.reshape(1, -1), pwa, pwb, pred_b.reshape(1, 1))
    return out.reshape(-1)


# speedup vs baseline: 1.6443x; 1.6443x over previous
import functools

import jax
import jax.numpy as jnp
from jax import lax
from jax.experimental import pallas as pl
from jax.experimental.pallas import tpu as pltpu
from jax.experimental.pallas import tpu_sc as plsc

BATCH = 16384
D_MLP = 256
D_GMF = 64
N_ROWS = 100000

_NC = 2
_NS = 16
_NW = _NC * _NS
_BPW = BATCH // _NW
_CHUNK = 64
_NCHUNK = _BPW // _CHUNK

_sc_mesh = plsc.VectorSubcoreMesh(core_axis_name="c", subcore_axis_name="s")


@functools.partial(
    pl.kernel,
    mesh=_sc_mesh,
    out_type=[
        jax.ShapeDtypeStruct((BATCH, D_MLP), jnp.float32),
        jax.ShapeDtypeStruct((BATCH, D_MLP), jnp.float32),
        jax.ShapeDtypeStruct((BATCH, D_GMF), jnp.float32),
    ],
    scratch_types=[
        pltpu.VMEM((_BPW,), jnp.int32),
        pltpu.VMEM((_BPW,), jnp.int32),
        pltpu.VMEM((2, _CHUNK, D_MLP), jnp.float32),
        pltpu.VMEM((2, _CHUNK, D_MLP), jnp.float32),
        pltpu.VMEM((2, _CHUNK, 2 * D_GMF), jnp.float32),
        pltpu.VMEM((2, _CHUNK, 2 * D_GMF), jnp.float32),
        pltpu.VMEM((_CHUNK, D_GMF), jnp.float32),
        pltpu.SemaphoreType.DMA,
        pltpu.SemaphoreType.DMA,
    ],
)
def _sc_gather(users_hbm, items_hbm, uemb_hbm, iemb_hbm, gcat_hbm,
               out_u, out_i, out_g,
               uidx_v, iidx_v, urows_v, irows_v, ucat_v, icat_v, g_v,
               sem0, sem1):
    wid = lax.axis_index("s") * _NC + lax.axis_index("c")
    base = wid * _BPW
    sems = (sem0, sem1)

    pltpu.sync_copy(users_hbm.at[pl.ds(base, _BPW)], uidx_v)
    pltpu.sync_copy(items_hbm.at[pl.ds(base, _BPW)], iidx_v)

    def fire(k):
        p = k % 2
        uix = uidx_v.at[pl.ds(k * _CHUNK, _CHUNK)]
        iix = iidx_v.at[pl.ds(k * _CHUNK, _CHUNK)]
        return (
            pltpu.async_copy(uemb_hbm.at[uix], urows_v.at[p], sems[p]),
            pltpu.async_copy(iemb_hbm.at[iix], irows_v.at[p], sems[p]),
            pltpu.async_copy(gcat_hbm.at[uix], ucat_v.at[p], sems[p]),
            pltpu.async_copy(gcat_hbm.at[iix], icat_v.at[p], sems[p]),
        )

    inflight = fire(0)
    for k in range(_NCHUNK):
        nxt = fire(k + 1) if k + 1 < _NCHUNK else None
        for c in inflight:
            c.wait()
        p = k % 2
        off = base + k * _CHUNK

        def mul_body(r, mc):
            for c in range(D_GMF // 16):
                s = pl.ds(c * 16, 16)
                s_hi = pl.ds(D_GMF + c * 16, 16)
                g_v[r, s] = ucat_v[p, r, s] * icat_v[p, r, s_hi]
            return mc

        lax.fori_loop(0, _CHUNK, mul_body, 0)

        pltpu.sync_copy(urows_v.at[p], out_u.at[pl.ds(off, _CHUNK)])
        pltpu.sync_copy(irows_v.at[p], out_i.at[pl.ds(off, _CHUNK)])
        pltpu.sync_copy(g_v, out_g.at[pl.ds(off, _CHUNK)])
        inflight = nxt


def _tc_body(ut_ref, it_ref, o_ref):
    o_ref[:, 0:D_GMF] = ut_ref[...].T
    o_ref[:, D_GMF:2 * D_GMF] = it_ref[...].T


def _transpose_concat(ut, it, block_n=2048):
    grid = (pl.cdiv(N_ROWS, block_n),)
    return pl.pallas_call(
        _tc_body,
        grid=grid,
        in_specs=[
            pl.BlockSpec((D_GMF, block_n), lambda m: (0, m)),
            pl.BlockSpec((D_GMF, block_n), lambda m: (0, m)),
        ],
        out_specs=pl.BlockSpec((block_n, 2 * D_GMF), lambda m: (m, 0)),
        out_shape=jax.ShapeDtypeStruct((N_ROWS, 2 * D_GMF), jnp.float32),
        compiler_params=pltpu.CompilerParams(
            dimension_semantics=("arbitrary",),
        ),
    )(ut, it)


def _dense_body(u_ref, i_ref, g_ref, w0a_ref, w0b_ref, b0_ref, w1_ref,
                b1_ref, w2_ref, b2_ref, pwa_ref, pwb_ref, pb_ref, o_ref):
    h = jnp.dot(u_ref[...], w0a_ref[...], preferred_element_type=jnp.float32)
    h += jnp.dot(i_ref[...], w0b_ref[...], preferred_element_type=jnp.float32)
    h = jnp.maximum(h + b0_ref[...], 0.0)
    h = jnp.dot(h, w1_ref[...], preferred_element_type=jnp.float32)
    h = jnp.maximum(h + b1_ref[...], 0.0)
    h = jnp.dot(h, w2_ref[...], preferred_element_type=jnp.float32)
    h = jnp.maximum(h + b2_ref[...], 0.0)
    logit = jnp.dot(h, pwa_ref[...], preferred_element_type=jnp.float32)
    logit += jnp.dot(g_ref[...], pwb_ref[...], preferred_element_type=jnp.float32)
    logit += pb_ref[0, 0]
    o_ref[...] = 1.0 / (1.0 + jnp.exp(-logit))


def _dense(u_rows, i_rows, g, w0a, w0b, b0, w1, b1, w2, b2, pwa, pwb, pb,
           block_m=2048):
    grid = (BATCH // block_m,)
    full = lambda m: (0, 0)
    return pl.pallas_call(
        _dense_body,
        grid=grid,
        in_specs=[
            pl.BlockSpec((block_m, D_MLP), lambda m: (m, 0)),
            pl.BlockSpec((block_m, D_MLP), lambda m: (m, 0)),
            pl.BlockSpec((block_m, D_GMF), lambda m: (m, 0)),
            pl.BlockSpec((D_MLP, 256), full),
            pl.BlockSpec((D_MLP, 256), full),
            pl.BlockSpec((1, 256), full),
            pl.BlockSpec((256, 128), full),
            pl.BlockSpec((1, 128), full),
            pl.BlockSpec((128, 64), full),
            pl.BlockSpec((1, 64), full),
            pl.BlockSpec((64, 1), full),
            pl.BlockSpec((64, 1), full),
            pl.BlockSpec((1, 1), full),
        ],
        out_specs=pl.BlockSpec((block_m, 1), lambda m: (m, 0)),
        out_shape=jax.ShapeDtypeStruct((BATCH, 1), jnp.float32),
        compiler_params=pltpu.CompilerParams(
            dimension_semantics=("arbitrary",),
        ),
    )(u_rows, i_rows, g, w0a, w0b, b0, w1, b1, w2, b2, pwa, pwb, pb)


def kernel(users, items, user_emb_mlp, item_emb_mlp, user_emb_gmf,
           item_emb_gmf, mlp_w0, mlp_b0, mlp_w1, mlp_b1, mlp_w2, mlp_b2,
           pred_w, pred_b):
    users = users.astype(jnp.int32)
    items = items.astype(jnp.int32)

    gmf_cat = _transpose_concat(user_emb_gmf.T, item_emb_gmf.T)
    u_rows, i_rows, g = _sc_gather(users, items, user_emb_mlp, item_emb_mlp,
                                   gmf_cat)

    w0a = mlp_w0[:D_MLP]
    w0b = mlp_w0[D_MLP:]
    pwa = pred_w[:D_GMF]
    pwb = pred_w[D_GMF:]
    out = _dense(u_rows, i_rows, g, w0a, w0b, mlp_b0.reshape(1, -1),
                 mlp_w1, mlp_b1.reshape(1, -1), mlp_w2,
                 mlp_b2.reshape(1, -1), pwa, pwb, pred_b.reshape(1, 1))
    return out.reshape(-1)


# speedup vs baseline: 1.8098x; 1.1007x over previous
import functools

import jax
import jax.numpy as jnp
from jax import lax
from jax.experimental import pallas as pl
from jax.experimental.pallas import tpu as pltpu
from jax.experimental.pallas import tpu_sc as plsc

BATCH = 16384
D_MLP = 256
D_GMF = 64
N_ROWS = 100000

_NC = 2
_NS = 16
_NW = _NC * _NS
_BPW = BATCH // _NW
_CHUNK = 64
_NCHUNK = _BPW // _CHUNK

_sc_mesh = plsc.VectorSubcoreMesh(core_axis_name="c", subcore_axis_name="s")


@functools.partial(
    pl.kernel,
    mesh=_sc_mesh,
    out_type=[
        jax.ShapeDtypeStruct((BATCH, D_MLP), jnp.float32),
        jax.ShapeDtypeStruct((BATCH, D_MLP), jnp.float32),
        jax.ShapeDtypeStruct((BATCH, D_GMF), jnp.float32),
    ],
    scratch_types=[
        pltpu.VMEM((_BPW,), jnp.int32),
        pltpu.VMEM((_BPW,), jnp.int32),
        pltpu.VMEM((2, _CHUNK, D_MLP), jnp.float32),
        pltpu.VMEM((2, _CHUNK, D_MLP), jnp.float32),
        pltpu.VMEM((2, _CHUNK, 2 * D_GMF), jnp.float32),
        pltpu.VMEM((2, _CHUNK, 2 * D_GMF), jnp.float32),
        pltpu.VMEM((_CHUNK, D_GMF), jnp.float32),
        pltpu.SemaphoreType.DMA,
        pltpu.SemaphoreType.DMA,
    ],
)
def _sc_gather(users_hbm, items_hbm, uemb_hbm, iemb_hbm, gcat_hbm,
               out_u, out_i, out_g,
               uidx_v, iidx_v, urows_v, irows_v, ucat_v, icat_v, g_v,
               sem0, sem1):
    wid = lax.axis_index("s") * _NC + lax.axis_index("c")
    base = wid * _BPW
    sems = (sem0, sem1)

    pltpu.sync_copy(users_hbm.at[pl.ds(base, _BPW)], uidx_v)
    pltpu.sync_copy(items_hbm.at[pl.ds(base, _BPW)], iidx_v)

    def fire(k):
        p = k % 2
        uix = uidx_v.at[pl.ds(k * _CHUNK, _CHUNK)]
        iix = iidx_v.at[pl.ds(k * _CHUNK, _CHUNK)]
        return (
            pltpu.async_copy(uemb_hbm.at[uix], urows_v.at[p], sems[p]),
            pltpu.async_copy(iemb_hbm.at[iix], irows_v.at[p], sems[p]),
            pltpu.async_copy(gcat_hbm.at[uix], ucat_v.at[p], sems[p]),
            pltpu.async_copy(gcat_hbm.at[iix], icat_v.at[p], sems[p]),
        )

    inflight = fire(0)
    for k in range(_NCHUNK):
        nxt = fire(k + 1) if k + 1 < _NCHUNK else None
        for c in inflight:
            c.wait()
        p = k % 2
        off = base + k * _CHUNK

        def mul_body(r, mc):
            for c in range(D_GMF // 16):
                s = pl.ds(c * 16, 16)
                s_hi = pl.ds(D_GMF + c * 16, 16)
                g_v[r, s] = ucat_v[p, r, s] * icat_v[p, r, s_hi]
            return mc

        lax.fori_loop(0, _CHUNK, mul_body, 0)

        pltpu.sync_copy(urows_v.at[p], out_u.at[pl.ds(off, _CHUNK)])
        pltpu.sync_copy(irows_v.at[p], out_i.at[pl.ds(off, _CHUNK)])
        pltpu.sync_copy(g_v, out_g.at[pl.ds(off, _CHUNK)])
        inflight = nxt


def _tc_body(ut_ref, it_ref, o_ref):
    eye = jnp.eye(D_GMF, dtype=jnp.float32)
    dn = (((0,), (0,)), ((), ()))
    o_ref[:, 0:D_GMF] = lax.dot_general(
        ut_ref[...], eye, dn, preferred_element_type=jnp.float32)
    o_ref[:, D_GMF:2 * D_GMF] = lax.dot_general(
        it_ref[...], eye, dn, preferred_element_type=jnp.float32)


def _transpose_concat(ut, it, block_n=4096):
    grid = (pl.cdiv(N_ROWS, block_n),)
    return pl.pallas_call(
        _tc_body,
        grid=grid,
        in_specs=[
            pl.BlockSpec((D_GMF, block_n), lambda m: (0, m)),
            pl.BlockSpec((D_GMF, block_n), lambda m: (0, m)),
        ],
        out_specs=pl.BlockSpec((block_n, 2 * D_GMF), lambda m: (m, 0)),
        out_shape=jax.ShapeDtypeStruct((N_ROWS, 2 * D_GMF), jnp.float32),
        compiler_params=pltpu.CompilerParams(
            dimension_semantics=("arbitrary",),
        ),
    )(ut, it)


def _dense_body(u_ref, i_ref, g_ref, w0a_ref, w0b_ref, b0_ref, w1_ref,
                b1_ref, w2_ref, b2_ref, pwa_ref, pwb_ref, pb_ref, o_ref):
    h = jnp.dot(u_ref[...], w0a_ref[...], preferred_element_type=jnp.float32)
    h += jnp.dot(i_ref[...], w0b_ref[...], preferred_element_type=jnp.float32)
    h = jnp.maximum(h + b0_ref[...], 0.0)
    h = jnp.dot(h, w1_ref[...], preferred_element_type=jnp.float32)
    h = jnp.maximum(h + b1_ref[...], 0.0)
    h = jnp.dot(h, w2_ref[...], preferred_element_type=jnp.float32)
    h = jnp.maximum(h + b2_ref[...], 0.0)
    logit = jnp.dot(h, pwa_ref[...], preferred_element_type=jnp.float32)
    logit += jnp.dot(g_ref[...], pwb_ref[...], preferred_element_type=jnp.float32)
    logit += pb_ref[0, 0]
    o_ref[...] = (1.0 / (1.0 + jnp.exp(-logit)))[:, 0]


def _dense(u_rows, i_rows, g, w0a, w0b, b0, w1, b1, w2, b2, pwa, pwb, pb,
           block_m=2048):
    grid = (BATCH // block_m,)
    full = lambda m: (0, 0)
    return pl.pallas_call(
        _dense_body,
        grid=grid,
        in_specs=[
            pl.BlockSpec((block_m, D_MLP), lambda m: (m, 0)),
            pl.BlockSpec((block_m, D_MLP), lambda m: (m, 0)),
            pl.BlockSpec((block_m, D_GMF), lambda m: (m, 0)),
            pl.BlockSpec((D_MLP, 256), full),
            pl.BlockSpec((D_MLP, 256), full),
            pl.BlockSpec((1, 256), full),
            pl.BlockSpec((256, 128), full),
            pl.BlockSpec((1, 128), full),
            pl.BlockSpec((128, 64), full),
            pl.BlockSpec((1, 64), full),
            pl.BlockSpec((64, 1), full),
            pl.BlockSpec((64, 1), full),
            pl.BlockSpec((1, 1), full),
        ],
        out_specs=pl.BlockSpec((block_m,), lambda m: (m,)),
        out_shape=jax.ShapeDtypeStruct((BATCH,), jnp.float32),
        compiler_params=pltpu.CompilerParams(
            dimension_semantics=("arbitrary",),
        ),
    )(u_rows, i_rows, g, w0a, w0b, b0, w1, b1, w2, b2, pwa, pwb, pb)


def kernel(users, items, user_emb_mlp, item_emb_mlp, user_emb_gmf,
           item_emb_gmf, mlp_w0, mlp_b0, mlp_w1, mlp_b1, mlp_w2, mlp_b2,
           pred_w, pred_b):
    users = users.astype(jnp.int32)
    items = items.astype(jnp.int32)

    gmf_cat = _transpose_concat(user_emb_gmf.T, item_emb_gmf.T)
    u_rows, i_rows, g = _sc_gather(users, items, user_emb_mlp, item_emb_mlp,
                                   gmf_cat)

    w0a = mlp_w0[:D_MLP]
    w0b = mlp_w0[D_MLP:]
    pwa = pred_w[:D_GMF]
    pwb = pred_w[D_GMF:]
    out = _dense(u_rows, i_rows, g, w0a, w0b, mlp_b0.reshape(1, -1),
                 mlp_w1, mlp_b1.reshape(1, -1), mlp_w2,
                 mlp_b2.reshape(1, -1), pwa, pwb, pred_b.reshape(1, 1))
    return out


# speedup vs baseline: 1.8626x; 1.0292x over previous
import functools

import jax
import jax.numpy as jnp
from jax import lax
from jax.experimental import pallas as pl
from jax.experimental.pallas import tpu as pltpu
from jax.experimental.pallas import tpu_sc as plsc

BATCH = 16384
D_MLP = 256
D_GMF = 64
N_ROWS = 100000

_NC = 2
_NS = 16
_NW = _NC * _NS
_BPW = BATCH // _NW
_CHUNK = 64
_NCHUNK = _BPW // _CHUNK

_sc_mesh = plsc.VectorSubcoreMesh(core_axis_name="c", subcore_axis_name="s")


@functools.partial(
    pl.kernel,
    mesh=_sc_mesh,
    out_type=[
        jax.ShapeDtypeStruct((BATCH, D_MLP), jnp.float32),
        jax.ShapeDtypeStruct((BATCH, D_MLP), jnp.float32),
        jax.ShapeDtypeStruct((BATCH, D_GMF), jnp.float32),
    ],
    scratch_types=[
        pltpu.VMEM((_BPW,), jnp.int32),
        pltpu.VMEM((_BPW,), jnp.int32),
        pltpu.VMEM((2, _CHUNK, D_MLP), jnp.float32),
        pltpu.VMEM((2, _CHUNK, D_MLP), jnp.float32),
        pltpu.VMEM((2, _CHUNK, 2 * D_GMF), jnp.float32),
        pltpu.VMEM((2, _CHUNK, 2 * D_GMF), jnp.float32),
        pltpu.VMEM((_CHUNK, D_GMF), jnp.float32),
        pltpu.SemaphoreType.DMA,
        pltpu.SemaphoreType.DMA,
    ],
)
def _sc_gather(users_hbm, items_hbm, uemb_hbm, iemb_hbm, gcat_hbm,
               out_u, out_i, out_g,
               uidx_v, iidx_v, urows_v, irows_v, ucat_v, icat_v, g_v,
               sem0, sem1):
    wid = lax.axis_index("s") * _NC + lax.axis_index("c")
    base = wid * _BPW
    sems = (sem0, sem1)

    pltpu.sync_copy(users_hbm.at[pl.ds(base, _BPW)], uidx_v)
    pltpu.sync_copy(items_hbm.at[pl.ds(base, _BPW)], iidx_v)

    def fire(k):
        p = k % 2
        uix = uidx_v.at[pl.ds(k * _CHUNK, _CHUNK)]
        iix = iidx_v.at[pl.ds(k * _CHUNK, _CHUNK)]
        return (
            pltpu.async_copy(uemb_hbm.at[uix], urows_v.at[p], sems[p]),
            pltpu.async_copy(iemb_hbm.at[iix], irows_v.at[p], sems[p]),
            pltpu.async_copy(gcat_hbm.at[uix], ucat_v.at[p], sems[p]),
            pltpu.async_copy(gcat_hbm.at[iix], icat_v.at[p], sems[p]),
        )

    inflight = fire(0)
    for k in range(_NCHUNK):
        nxt = fire(k + 1) if k + 1 < _NCHUNK else None
        for c in inflight:
            c.wait()
        p = k % 2
        off = base + k * _CHUNK

        def mul_body(r, mc):
            for c in range(D_GMF // 16):
                s = pl.ds(c * 16, 16)
                s_hi = pl.ds(D_GMF + c * 16, 16)
                g_v[r, s] = ucat_v[p, r, s] * icat_v[p, r, s_hi]
            return mc

        lax.fori_loop(0, _CHUNK, mul_body, 0)

        pltpu.sync_copy(urows_v.at[p], out_u.at[pl.ds(off, _CHUNK)])
        pltpu.sync_copy(irows_v.at[p], out_i.at[pl.ds(off, _CHUNK)])
        pltpu.sync_copy(g_v, out_g.at[pl.ds(off, _CHUNK)])
        inflight = nxt


def _tc_body(ut_ref, it_ref, o_ref):
    eye = jnp.eye(D_GMF, dtype=jnp.float32)
    dn = (((0,), (0,)), ((), ()))
    o_ref[:, 0:D_GMF] = lax.dot_general(
        ut_ref[...], eye, dn, preferred_element_type=jnp.float32)
    o_ref[:, D_GMF:2 * D_GMF] = lax.dot_general(
        it_ref[...], eye, dn, preferred_element_type=jnp.float32)


def _transpose_concat(ut, it, block_n=8192):
    grid = (pl.cdiv(N_ROWS, block_n),)
    return pl.pallas_call(
        _tc_body,
        grid=grid,
        in_specs=[
            pl.BlockSpec((D_GMF, block_n), lambda m: (0, m)),
            pl.BlockSpec((D_GMF, block_n), lambda m: (0, m)),
        ],
        out_specs=pl.BlockSpec((block_n, 2 * D_GMF), lambda m: (m, 0)),
        out_shape=jax.ShapeDtypeStruct((N_ROWS, 2 * D_GMF), jnp.float32),
        compiler_params=pltpu.CompilerParams(
            dimension_semantics=("arbitrary",),
        ),
    )(ut, it)


def _dense_body(u_ref, i_ref, g_ref, w0a_ref, w0b_ref, b0_ref, w1_ref,
                b1_ref, w2_ref, b2_ref, pwa_ref, pwb_ref, pb_ref, o_ref):
    h = jnp.dot(u_ref[...], w0a_ref[...], preferred_element_type=jnp.float32)
    h += jnp.dot(i_ref[...], w0b_ref[...], preferred_element_type=jnp.float32)
    h = jnp.maximum(h + b0_ref[...], 0.0)
    h = jnp.dot(h, w1_ref[...], preferred_element_type=jnp.float32)
    h = jnp.maximum(h + b1_ref[...], 0.0)
    h = jnp.dot(h, w2_ref[...], preferred_element_type=jnp.float32)
    h = jnp.maximum(h + b2_ref[...], 0.0)
    logit = jnp.dot(h, pwa_ref[...], preferred_element_type=jnp.float32)
    logit += jnp.dot(g_ref[...], pwb_ref[...], preferred_element_type=jnp.float32)
    logit += pb_ref[0, 0]
    o_ref[...] = (1.0 / (1.0 + jnp.exp(-logit)))[:, 0]


def _dense(u_rows, i_rows, g, w0a, w0b, b0, w1, b1, w2, b2, pwa, pwb, pb,
           block_m=4096):
    grid = (BATCH // block_m,)
    full = lambda m: (0, 0)
    return pl.pallas_call(
        _dense_body,
        grid=grid,
        in_specs=[
            pl.BlockSpec((block_m, D_MLP), lambda m: (m, 0)),
            pl.BlockSpec((block_m, D_MLP), lambda m: (m, 0)),
            pl.BlockSpec((block_m, D_GMF), lambda m: (m, 0)),
            pl.BlockSpec((D_MLP, 256), full),
            pl.BlockSpec((D_MLP, 256), full),
            pl.BlockSpec((1, 256), full),
            pl.BlockSpec((256, 128), full),
            pl.BlockSpec((1, 128), full),
            pl.BlockSpec((128, 64), full),
            pl.BlockSpec((1, 64), full),
            pl.BlockSpec((64, 1), full),
            pl.BlockSpec((64, 1), full),
            pl.BlockSpec((1, 1), full),
        ],
        out_specs=pl.BlockSpec((block_m,), lambda m: (m,)),
        out_shape=jax.ShapeDtypeStruct((BATCH,), jnp.float32),
        compiler_params=pltpu.CompilerParams(
            dimension_semantics=("arbitrary",),
        ),
    )(u_rows, i_rows, g, w0a, w0b, b0, w1, b1, w2, b2, pwa, pwb, pb)


def kernel(users, items, user_emb_mlp, item_emb_mlp, user_emb_gmf,
           item_emb_gmf, mlp_w0, mlp_b0, mlp_w1, mlp_b1, mlp_w2, mlp_b2,
           pred_w, pred_b):
    users = users.astype(jnp.int32)
    items = items.astype(jnp.int32)

    gmf_cat = _transpose_concat(user_emb_gmf.T, item_emb_gmf.T)
    u_rows, i_rows, g = _sc_gather(users, items, user_emb_mlp, item_emb_mlp,
                                   gmf_cat)

    w0a = mlp_w0[:D_MLP]
    w0b = mlp_w0[D_MLP:]
    pwa = pred_w[:D_GMF]
    pwb = pred_w[D_GMF:]
    out = _dense(u_rows, i_rows, g, w0a, w0b, mlp_b0.reshape(1, -1),
                 mlp_w1, mlp_b1.reshape(1, -1), mlp_w2,
                 mlp_b2.reshape(1, -1), pwa, pwb, pred_b.reshape(1, 1))
    return out


# speedup vs baseline: 1.9684x; 1.0568x over previous
import functools

import jax
import jax.numpy as jnp
from jax import lax
from jax.experimental import pallas as pl
from jax.experimental.pallas import tpu as pltpu
from jax.experimental.pallas import tpu_sc as plsc

BATCH = 16384
D_MLP = 256
D_GMF = 64
N_ROWS = 100000

_NC = 2
_NS = 16
_NW = _NC * _NS
_BPW = BATCH // _NW
_CHUNK = 64
_NCHUNK = _BPW // _CHUNK

_sc_mesh = plsc.VectorSubcoreMesh(core_axis_name="c", subcore_axis_name="s")


@functools.partial(
    pl.kernel,
    mesh=_sc_mesh,
    out_type=[
        jax.ShapeDtypeStruct((BATCH, D_MLP), jnp.float32),
        jax.ShapeDtypeStruct((BATCH, D_MLP), jnp.float32),
    ],
    scratch_types=[
        pltpu.VMEM((_BPW,), jnp.int32),
        pltpu.VMEM((_BPW,), jnp.int32),
        pltpu.VMEM((2, _CHUNK, D_MLP), jnp.float32),
        pltpu.VMEM((2, _CHUNK, D_MLP), jnp.float32),
        pltpu.SemaphoreType.DMA,
        pltpu.SemaphoreType.DMA,
    ],
)
def _sc_gather_mlp(users_hbm, items_hbm, uemb_hbm, iemb_hbm,
                   out_u, out_i,
                   uidx_v, iidx_v, urows_v, irows_v, sem0, sem1):
    wid = lax.axis_index("s") * _NC + lax.axis_index("c")
    base = wid * _BPW
    sems = (sem0, sem1)

    pltpu.sync_copy(users_hbm.at[pl.ds(base, _BPW)], uidx_v)
    pltpu.sync_copy(items_hbm.at[pl.ds(base, _BPW)], iidx_v)

    def fire(k):
        p = k % 2
        uix = uidx_v.at[pl.ds(k * _CHUNK, _CHUNK)]
        iix = iidx_v.at[pl.ds(k * _CHUNK, _CHUNK)]
        return (
            pltpu.async_copy(uemb_hbm.at[uix], urows_v.at[p], sems[p]),
            pltpu.async_copy(iemb_hbm.at[iix], irows_v.at[p], sems[p]),
        )

    inflight = fire(0)
    for k in range(_NCHUNK):
        nxt = fire(k + 1) if k + 1 < _NCHUNK else None
        for c in inflight:
            c.wait()
        p = k % 2
        off = base + k * _CHUNK
        pltpu.sync_copy(urows_v.at[p], out_u.at[pl.ds(off, _CHUNK)])
        pltpu.sync_copy(irows_v.at[p], out_i.at[pl.ds(off, _CHUNK)])
        inflight = nxt


@functools.partial(
    pl.kernel,
    mesh=_sc_mesh,
    out_type=[
        jax.ShapeDtypeStruct((BATCH, D_GMF), jnp.float32),
    ],
    scratch_types=[
        pltpu.VMEM((_BPW,), jnp.int32),
        pltpu.VMEM((_BPW,), jnp.int32),
        pltpu.VMEM((2, _CHUNK, 2 * D_GMF), jnp.float32),
        pltpu.VMEM((2, _CHUNK, 2 * D_GMF), jnp.float32),
        pltpu.VMEM((_CHUNK, D_GMF), jnp.float32),
        pltpu.SemaphoreType.DMA,
        pltpu.SemaphoreType.DMA,
    ],
)
def _sc_gather_gmf(users_hbm, items_hbm, gcat_hbm, out_g,
                   uidx_v, iidx_v, ucat_v, icat_v, g_v, sem0, sem1):
    wid = lax.axis_index("s") * _NC + lax.axis_index("c")
    base = wid * _BPW
    sems = (sem0, sem1)

    pltpu.sync_copy(users_hbm.at[pl.ds(base, _BPW)], uidx_v)
    pltpu.sync_copy(items_hbm.at[pl.ds(base, _BPW)], iidx_v)

    def fire(k):
        p = k % 2
        uix = uidx_v.at[pl.ds(k * _CHUNK, _CHUNK)]
        iix = iidx_v.at[pl.ds(k * _CHUNK, _CHUNK)]
        return (
            pltpu.async_copy(gcat_hbm.at[uix], ucat_v.at[p], sems[p]),
            pltpu.async_copy(gcat_hbm.at[iix], icat_v.at[p], sems[p]),
        )

    inflight = fire(0)
    for k in range(_NCHUNK):
        nxt = fire(k + 1) if k + 1 < _NCHUNK else None
        for c in inflight:
            c.wait()
        p = k % 2
        off = base + k * _CHUNK

        def mul_body(r, mc):
            for c in range(D_GMF // 16):
                s = pl.ds(c * 16, 16)
                s_hi = pl.ds(D_GMF + c * 16, 16)
                g_v[r, s] = ucat_v[p, r, s] * icat_v[p, r, s_hi]
            return mc

        lax.fori_loop(0, _CHUNK, mul_body, 0)
        pltpu.sync_copy(g_v, out_g.at[pl.ds(off, _CHUNK)])
        inflight = nxt


def _tc_body(ut_ref, it_ref, o_ref):
    eye = jnp.eye(D_GMF, dtype=jnp.float32)
    dn = (((0,), (0,)), ((), ()))
    o_ref[:, 0:D_GMF] = lax.dot_general(
        ut_ref[...], eye, dn, preferred_element_type=jnp.float32)
    o_ref[:, D_GMF:2 * D_GMF] = lax.dot_general(
        it_ref[...], eye, dn, preferred_element_type=jnp.float32)


def _transpose_concat(ut, it, block_n=8192):
    grid = (pl.cdiv(N_ROWS, block_n),)
    return pl.pallas_call(
        _tc_body,
        grid=grid,
        in_specs=[
            pl.BlockSpec((D_GMF, block_n), lambda m: (0, m)),
            pl.BlockSpec((D_GMF, block_n), lambda m: (0, m)),
        ],
        out_specs=pl.BlockSpec((block_n, 2 * D_GMF), lambda m: (m, 0)),
        out_shape=jax.ShapeDtypeStruct((N_ROWS, 2 * D_GMF), jnp.float32),
        compiler_params=pltpu.CompilerParams(
            dimension_semantics=("arbitrary",),
        ),
    )(ut, it)


def _dense_body(u_ref, i_ref, g_ref, w0a_ref, w0b_ref, b0_ref, w1_ref,
                b1_ref, w2_ref, b2_ref, pwa_ref, pwb_ref, pb_ref, o_ref):
    h = jnp.dot(u_ref[...], w0a_ref[...], preferred_element_type=jnp.float32)
    h += jnp.dot(i_ref[...], w0b_ref[...], preferred_element_type=jnp.float32)
    h = jnp.maximum(h + b0_ref[...], 0.0)
    h = jnp.dot(h, w1_ref[...], preferred_element_type=jnp.float32)
    h = jnp.maximum(h + b1_ref[...], 0.0)
    h = jnp.dot(h, w2_ref[...], preferred_element_type=jnp.float32)
    h = jnp.maximum(h + b2_ref[...], 0.0)
    logit = jnp.dot(h, pwa_ref[...], preferred_element_type=jnp.float32)
    logit += jnp.dot(g_ref[...], pwb_ref[...], preferred_element_type=jnp.float32)
    logit += pb_ref[0, 0]
    o_ref[...] = (1.0 / (1.0 + jnp.exp(-logit)))[:, 0]


def _dense(u_rows, i_rows, g, w0a, w0b, b0, w1, b1, w2, b2, pwa, pwb, pb,
           block_m=4096):
    grid = (BATCH // block_m,)
    full = lambda m: (0, 0)
    return pl.pallas_call(
        _dense_body,
        grid=grid,
        in_specs=[
            pl.BlockSpec((block_m, D_MLP), lambda m: (m, 0)),
            pl.BlockSpec((block_m, D_MLP), lambda m: (m, 0)),
            pl.BlockSpec((block_m, D_GMF), lambda m: (m, 0)),
            pl.BlockSpec((D_MLP, 256), full),
            pl.BlockSpec((D_MLP, 256), full),
            pl.BlockSpec((1, 256), full),
            pl.BlockSpec((256, 128), full),
            pl.BlockSpec((1, 128), full),
            pl.BlockSpec((128, 64), full),
            pl.BlockSpec((1, 64), full),
            pl.BlockSpec((64, 1), full),
            pl.BlockSpec((64, 1), full),
            pl.BlockSpec((1, 1), full),
        ],
        out_specs=pl.BlockSpec((block_m,), lambda m: (m,)),
        out_shape=jax.ShapeDtypeStruct((BATCH,), jnp.float32),
        compiler_params=pltpu.CompilerParams(
            dimension_semantics=("arbitrary",),
        ),
    )(u_rows, i_rows, g, w0a, w0b, b0, w1, b1, w2, b2, pwa, pwb, pb)


def kernel(users, items, user_emb_mlp, item_emb_mlp, user_emb_gmf,
           item_emb_gmf, mlp_w0, mlp_b0, mlp_w1, mlp_b1, mlp_w2, mlp_b2,
           pred_w, pred_b):
    users = users.astype(jnp.int32)
    items = items.astype(jnp.int32)

    u_rows, i_rows = _sc_gather_mlp(users, items, user_emb_mlp, item_emb_mlp)
    gmf_cat = _transpose_concat(user_emb_gmf.T, item_emb_gmf.T)
    (g,) = _sc_gather_gmf(users, items, gmf_cat)

    w0a = mlp_w0[:D_MLP]
    w0b = mlp_w0[D_MLP:]
    pwa = pred_w[:D_GMF]
    pwb = pred_w[D_GMF:]
    out = _dense(u_rows, i_rows, g, w0a, w0b, mlp_b0.reshape(1, -1),
                 mlp_w1, mlp_b1.reshape(1, -1), mlp_w2,
                 mlp_b2.reshape(1, -1), pwa, pwb, pred_b.reshape(1, 1))
    return out


# speedup vs baseline: 1.9906x; 1.0113x over previous
import functools

import jax
import jax.numpy as jnp
from jax import lax
from jax.experimental import pallas as pl
from jax.experimental.pallas import tpu as pltpu
from jax.experimental.pallas import tpu_sc as plsc

BATCH = 16384
D_MLP = 256
D_GMF = 64
N_ROWS = 100000

_NC = 2
_NS = 16
_NW = _NC * _NS
_BPW = BATCH // _NW
_CHUNK = 64
_NCHUNK = _BPW // _CHUNK

_sc_mesh = plsc.VectorSubcoreMesh(core_axis_name="c", subcore_axis_name="s")


@functools.partial(
    pl.kernel,
    mesh=_sc_mesh,
    out_type=[
        jax.ShapeDtypeStruct((BATCH, D_MLP), jnp.float32),
        jax.ShapeDtypeStruct((BATCH, D_MLP), jnp.float32),
    ],
    scratch_types=[
        pltpu.VMEM((_BPW,), jnp.int32),
        pltpu.VMEM((_BPW,), jnp.int32),
        pltpu.VMEM((2, _CHUNK, D_MLP), jnp.float32),
        pltpu.VMEM((2, _CHUNK, D_MLP), jnp.float32),
        pltpu.SemaphoreType.DMA,
        pltpu.SemaphoreType.DMA,
    ],
)
def _sc_gather_mlp(users_hbm, items_hbm, uemb_hbm, iemb_hbm,
                   out_u, out_i,
                   uidx_v, iidx_v, urows_v, irows_v, sem0, sem1):
    wid = lax.axis_index("s") * _NC + lax.axis_index("c")
    base = wid * _BPW
    sems = (sem0, sem1)

    pltpu.sync_copy(users_hbm.at[pl.ds(base, _BPW)], uidx_v)
    pltpu.sync_copy(items_hbm.at[pl.ds(base, _BPW)], iidx_v)

    def fire(k):
        p = k % 2
        uix = uidx_v.at[pl.ds(k * _CHUNK, _CHUNK)]
        iix = iidx_v.at[pl.ds(k * _CHUNK, _CHUNK)]
        return (
            pltpu.async_copy(uemb_hbm.at[uix], urows_v.at[p], sems[p]),
            pltpu.async_copy(iemb_hbm.at[iix], irows_v.at[p], sems[p]),
        )

    inflight = fire(0)
    for k in range(_NCHUNK):
        nxt = fire(k + 1) if k + 1 < _NCHUNK else None
        for c in inflight:
            c.wait()
        p = k % 2
        off = base + k * _CHUNK
        pltpu.sync_copy(urows_v.at[p], out_u.at[pl.ds(off, _CHUNK)])
        pltpu.sync_copy(irows_v.at[p], out_i.at[pl.ds(off, _CHUNK)])
        inflight = nxt


@functools.partial(
    pl.kernel,
    mesh=_sc_mesh,
    out_type=[
        jax.ShapeDtypeStruct((BATCH, D_GMF), jnp.float32),
    ],
    scratch_types=[
        pltpu.VMEM((_BPW,), jnp.int32),
        pltpu.VMEM((_BPW,), jnp.int32),
        pltpu.VMEM((2, _CHUNK, 2 * D_GMF), jnp.float32),
        pltpu.VMEM((2, _CHUNK, 2 * D_GMF), jnp.float32),
        pltpu.VMEM((_CHUNK, D_GMF), jnp.float32),
        pltpu.SemaphoreType.DMA,
        pltpu.SemaphoreType.DMA,
    ],
)
def _sc_gather_gmf(users_hbm, items_hbm, gcat_hbm, out_g,
                   uidx_v, iidx_v, ucat_v, icat_v, g_v, sem0, sem1):
    wid = lax.axis_index("s") * _NC + lax.axis_index("c")
    base = wid * _BPW
    sems = (sem0, sem1)

    pltpu.sync_copy(users_hbm.at[pl.ds(base, _BPW)], uidx_v)
    pltpu.sync_copy(items_hbm.at[pl.ds(base, _BPW)], iidx_v)

    def fire(k):
        p = k % 2
        uix = uidx_v.at[pl.ds(k * _CHUNK, _CHUNK)]
        iix = iidx_v.at[pl.ds(k * _CHUNK, _CHUNK)]
        return (
            pltpu.async_copy(gcat_hbm.at[uix], ucat_v.at[p], sems[p]),
            pltpu.async_copy(gcat_hbm.at[iix], icat_v.at[p], sems[p]),
        )

    inflight = fire(0)
    for k in range(_NCHUNK):
        nxt = fire(k + 1) if k + 1 < _NCHUNK else None
        for c in inflight:
            c.wait()
        p = k % 2
        off = base + k * _CHUNK

        def mul_body(r, mc):
            for c in range(D_GMF // 16):
                s = pl.ds(c * 16, 16)
                s_hi = pl.ds(D_GMF + c * 16, 16)
                g_v[r, s] = ucat_v[p, r, s] * icat_v[p, r, s_hi]
            return mc

        lax.fori_loop(0, _CHUNK, mul_body, 0)
        pltpu.sync_copy(g_v, out_g.at[pl.ds(off, _CHUNK)])
        inflight = nxt


def _tc_body(ut_ref, it_ref, o_ref):
    eye = jnp.eye(D_GMF, dtype=jnp.float32)
    dn = (((0,), (0,)), ((), ()))
    o_ref[:, 0:D_GMF] = lax.dot_general(
        ut_ref[...], eye, dn, preferred_element_type=jnp.float32)
    o_ref[:, D_GMF:2 * D_GMF] = lax.dot_general(
        it_ref[...], eye, dn, preferred_element_type=jnp.float32)


def _transpose_concat(ut, it, block_n=8192):
    grid = (pl.cdiv(N_ROWS, block_n),)
    return pl.pallas_call(
        _tc_body,
        grid=grid,
        in_specs=[
            pl.BlockSpec((D_GMF, block_n), lambda m: (0, m)),
            pl.BlockSpec((D_GMF, block_n), lambda m: (0, m)),
        ],
        out_specs=pl.BlockSpec((block_n, 2 * D_GMF), lambda m: (m, 0)),
        out_shape=jax.ShapeDtypeStruct((N_ROWS, 2 * D_GMF), jnp.float32),
        compiler_params=pltpu.CompilerParams(
            dimension_semantics=("arbitrary",),
        ),
    )(ut, it)


def _mlp_body(u_ref, i_ref, w0a_ref, w0b_ref, b0_ref, w1_ref,
              b1_ref, w2_ref, b2_ref, o_ref):
    h = jnp.dot(u_ref[...], w0a_ref[...], preferred_element_type=jnp.float32)
    h += jnp.dot(i_ref[...], w0b_ref[...], preferred_element_type=jnp.float32)
    h = jnp.maximum(h + b0_ref[...], 0.0)
    h = jnp.dot(h, w1_ref[...], preferred_element_type=jnp.float32)
    h = jnp.maximum(h + b1_ref[...], 0.0)
    h = jnp.dot(h, w2_ref[...], preferred_element_type=jnp.float32)
    o_ref[...] = jnp.maximum(h + b2_ref[...], 0.0)


def _mlp(u_rows, i_rows, w0a, w0b, b0, w1, b1, w2, b2, block_m=4096):
    grid = (BATCH // block_m,)
    full = lambda m: (0, 0)
    return pl.pallas_call(
        _mlp_body,
        grid=grid,
        in_specs=[
            pl.BlockSpec((block_m, D_MLP), lambda m: (m, 0)),
            pl.BlockSpec((block_m, D_MLP), lambda m: (m, 0)),
            pl.BlockSpec((D_MLP, 256), full),
            pl.BlockSpec((D_MLP, 256), full),
            pl.BlockSpec((1, 256), full),
            pl.BlockSpec((256, 128), full),
            pl.BlockSpec((1, 128), full),
            pl.BlockSpec((128, 64), full),
            pl.BlockSpec((1, 64), full),
        ],
        out_specs=pl.BlockSpec((block_m, D_GMF), lambda m: (m, 0)),
        out_shape=jax.ShapeDtypeStruct((BATCH, D_GMF), jnp.float32),
        compiler_params=pltpu.CompilerParams(
            dimension_semantics=("arbitrary",),
        ),
    )(u_rows, i_rows, w0a, w0b, b0, w1, b1, w2, b2)


def _predict_body(h_ref, g_ref, pwa_ref, pwb_ref, pb_ref, o_ref):
    logit = jnp.dot(h_ref[...], pwa_ref[...],
                    preferred_element_type=jnp.float32)
    logit += jnp.dot(g_ref[...], pwb_ref[...],
                     preferred_element_type=jnp.float32)
    logit += pb_ref[0, 0]
    o_ref[...] = (1.0 / (1.0 + jnp.exp(-logit)))[:, 0]


def _predict(h3, g, pwa, pwb, pb, block_m=8192):
    grid = (BATCH // block_m,)
    full = lambda m: (0, 0)
    return pl.pallas_call(
        _predict_body,
        grid=grid,
        in_specs=[
            pl.BlockSpec((block_m, D_GMF), lambda m: (m, 0)),
            pl.BlockSpec((block_m, D_GMF), lambda m: (m, 0)),
            pl.BlockSpec((64, 1), full),
            pl.BlockSpec((64, 1), full),
            pl.BlockSpec((1, 1), full),
        ],
        out_specs=pl.BlockSpec((block_m,), lambda m: (m,)),
        out_shape=jax.ShapeDtypeStruct((BATCH,), jnp.float32),
        compiler_params=pltpu.CompilerParams(
            dimension_semantics=("arbitrary",),
        ),
    )(h3, g, pwa, pwb, pb)


def kernel(users, items, user_emb_mlp, item_emb_mlp, user_emb_gmf,
           item_emb_gmf, mlp_w0, mlp_b0, mlp_w1, mlp_b1, mlp_w2, mlp_b2,
           pred_w, pred_b):
    users = users.astype(jnp.int32)
    items = items.astype(jnp.int32)

    u_rows, i_rows = _sc_gather_mlp(users, items, user_emb_mlp, item_emb_mlp)
    gmf_cat = _transpose_concat(user_emb_gmf.T, item_emb_gmf.T)
    (g,) = _sc_gather_gmf(users, items, gmf_cat)

    w0a = mlp_w0[:D_MLP]
    w0b = mlp_w0[D_MLP:]
    pwa = pred_w[:D_GMF]
    pwb = pred_w[D_GMF:]
    h3 = _mlp(u_rows, i_rows, w0a, w0b, mlp_b0.reshape(1, -1),
              mlp_w1, mlp_b1.reshape(1, -1), mlp_w2, mlp_b2.reshape(1, -1))
    out = _predict(h3, g, pwa, pwb, pred_b.reshape(1, 1))
    return out


# speedup vs baseline: 1.9936x; 1.0015x over previous
import functools

import jax
import jax.numpy as jnp
from jax import lax
from jax.experimental import pallas as pl
from jax.experimental.pallas import tpu as pltpu
from jax.experimental.pallas import tpu_sc as plsc

BATCH = 16384
D_MLP = 256
D_GMF = 64
N_ROWS = 100000

_NC = 2
_NS = 16
_NW = _NC * _NS
_BPW = BATCH // _NW
_CHUNK = 64
_NCHUNK = _BPW // _CHUNK

_sc_mesh = plsc.VectorSubcoreMesh(core_axis_name="c", subcore_axis_name="s")


@functools.partial(
    pl.kernel,
    mesh=_sc_mesh,
    out_type=[
        jax.ShapeDtypeStruct((BATCH, D_MLP), jnp.float32),
        jax.ShapeDtypeStruct((BATCH, D_MLP), jnp.float32),
    ],
    scratch_types=[
        pltpu.VMEM((_BPW,), jnp.int32),
        pltpu.VMEM((_BPW,), jnp.int32),
        pltpu.VMEM((2, _CHUNK, D_MLP), jnp.float32),
        pltpu.VMEM((2, _CHUNK, D_MLP), jnp.float32),
        pltpu.SemaphoreType.DMA,
        pltpu.SemaphoreType.DMA,
    ],
)
def _sc_gather_mlp(users_hbm, items_hbm, uemb_hbm, iemb_hbm,
                   out_u, out_i,
                   uidx_v, iidx_v, urows_v, irows_v, sem0, sem1):
    wid = lax.axis_index("s") * _NC + lax.axis_index("c")
    base = wid * _BPW
    sems = (sem0, sem1)

    pltpu.sync_copy(users_hbm.at[pl.ds(base, _BPW)], uidx_v)
    pltpu.sync_copy(items_hbm.at[pl.ds(base, _BPW)], iidx_v)

    def fire(k):
        p = k % 2
        uix = uidx_v.at[pl.ds(k * _CHUNK, _CHUNK)]
        iix = iidx_v.at[pl.ds(k * _CHUNK, _CHUNK)]
        return (
            pltpu.async_copy(uemb_hbm.at[uix], urows_v.at[p], sems[p]),
            pltpu.async_copy(iemb_hbm.at[iix], irows_v.at[p], sems[p]),
        )

    inflight = fire(0)
    for k in range(_NCHUNK):
        nxt = fire(k + 1) if k + 1 < _NCHUNK else None
        for c in inflight:
            c.wait()
        p = k % 2
        off = base + k * _CHUNK
        pltpu.sync_copy(urows_v.at[p], out_u.at[pl.ds(off, _CHUNK)])
        pltpu.sync_copy(irows_v.at[p], out_i.at[pl.ds(off, _CHUNK)])
        inflight = nxt


@functools.partial(
    pl.kernel,
    mesh=_sc_mesh,
    out_type=[
        jax.ShapeDtypeStruct((BATCH, D_GMF), jnp.float32),
    ],
    scratch_types=[
        pltpu.VMEM((_BPW,), jnp.int32),
        pltpu.VMEM((_BPW,), jnp.int32),
        pltpu.VMEM((2, _CHUNK, 2 * D_GMF), jnp.float32),
        pltpu.VMEM((2, _CHUNK, 2 * D_GMF), jnp.float32),
        pltpu.VMEM((_CHUNK, D_GMF), jnp.float32),
        pltpu.SemaphoreType.DMA,
        pltpu.SemaphoreType.DMA,
    ],
)
def _sc_gather_gmf(users_hbm, items_hbm, gcat_hbm, out_g,
                   uidx_v, iidx_v, ucat_v, icat_v, g_v, sem0, sem1):
    wid = lax.axis_index("s") * _NC + lax.axis_index("c")
    base = wid * _BPW
    sems = (sem0, sem1)

    pltpu.sync_copy(users_hbm.at[pl.ds(base, _BPW)], uidx_v)
    pltpu.sync_copy(items_hbm.at[pl.ds(base, _BPW)], iidx_v)

    def fire(k):
        p = k % 2
        uix = uidx_v.at[pl.ds(k * _CHUNK, _CHUNK)]
        iix = iidx_v.at[pl.ds(k * _CHUNK, _CHUNK)]
        return (
            pltpu.async_copy(gcat_hbm.at[uix], ucat_v.at[p], sems[p]),
            pltpu.async_copy(gcat_hbm.at[iix], icat_v.at[p], sems[p]),
        )

    inflight = fire(0)
    for k in range(_NCHUNK):
        nxt = fire(k + 1) if k + 1 < _NCHUNK else None
        for c in inflight:
            c.wait()
        p = k % 2
        off = base + k * _CHUNK

        def mul_body(r, mc):
            for c in range(D_GMF // 16):
                s = pl.ds(c * 16, 16)
                s_hi = pl.ds(D_GMF + c * 16, 16)
                g_v[r, s] = ucat_v[p, r, s] * icat_v[p, r, s_hi]
            return mc

        lax.fori_loop(0, _CHUNK, mul_body, 0)
        pltpu.sync_copy(g_v, out_g.at[pl.ds(off, _CHUNK)])
        inflight = nxt


def _tc_body(ut_ref, it_ref, o_ref):
    eye = jnp.eye(D_GMF, dtype=jnp.float32)
    dn = (((0,), (0,)), ((), ()))
    o_ref[:, 0:D_GMF] = lax.dot_general(
        ut_ref[...], eye, dn, preferred_element_type=jnp.float32)
    o_ref[:, D_GMF:2 * D_GMF] = lax.dot_general(
        it_ref[...], eye, dn, preferred_element_type=jnp.float32)


def _transpose_concat(ut, it, block_n=12800):
    grid = (pl.cdiv(N_ROWS, block_n),)
    return pl.pallas_call(
        _tc_body,
        grid=grid,
        in_specs=[
            pl.BlockSpec((D_GMF, block_n), lambda m: (0, m)),
            pl.BlockSpec((D_GMF, block_n), lambda m: (0, m)),
        ],
        out_specs=pl.BlockSpec((block_n, 2 * D_GMF), lambda m: (m, 0)),
        out_shape=jax.ShapeDtypeStruct((N_ROWS, 2 * D_GMF), jnp.float32),
        compiler_params=pltpu.CompilerParams(
            dimension_semantics=("arbitrary",),
        ),
    )(ut, it)


def _mlp_body(u_ref, i_ref, w0a_ref, w0b_ref, b0_ref, w1_ref,
              b1_ref, w2_ref, b2_ref, o_ref):
    h = jnp.dot(u_ref[...], w0a_ref[...], preferred_element_type=jnp.float32)
    h += jnp.dot(i_ref[...], w0b_ref[...], preferred_element_type=jnp.float32)
    h = jnp.maximum(h + b0_ref[...], 0.0)
    h = jnp.dot(h, w1_ref[...], preferred_element_type=jnp.float32)
    h = jnp.maximum(h + b1_ref[...], 0.0)
    h = jnp.dot(h, w2_ref[...], preferred_element_type=jnp.float32)
    o_ref[...] = jnp.maximum(h + b2_ref[...], 0.0)


def _mlp(u_rows, i_rows, w0a, w0b, b0, w1, b1, w2, b2, block_m=4096):
    grid = (BATCH // block_m,)
    full = lambda m: (0, 0)
    return pl.pallas_call(
        _mlp_body,
        grid=grid,
        in_specs=[
            pl.BlockSpec((block_m, D_MLP), lambda m: (m, 0)),
            pl.BlockSpec((block_m, D_MLP), lambda m: (m, 0)),
            pl.BlockSpec((D_MLP, 256), full),
            pl.BlockSpec((D_MLP, 256), full),
            pl.BlockSpec((1, 256), full),
            pl.BlockSpec((256, 128), full),
            pl.BlockSpec((1, 128), full),
            pl.BlockSpec((128, 64), full),
            pl.BlockSpec((1, 64), full),
        ],
        out_specs=pl.BlockSpec((block_m, D_GMF), lambda m: (m, 0)),
        out_shape=jax.ShapeDtypeStruct((BATCH, D_GMF), jnp.float32),
        compiler_params=pltpu.CompilerParams(
            dimension_semantics=("arbitrary",),
        ),
    )(u_rows, i_rows, w0a, w0b, b0, w1, b1, w2, b2)


def _predict_body(h_ref, g_ref, pwa_ref, pwb_ref, pb_ref, o_ref):
    logit = jnp.dot(h_ref[...], pwa_ref[...],
                    preferred_element_type=jnp.float32)
    logit += jnp.dot(g_ref[...], pwb_ref[...],
                     preferred_element_type=jnp.float32)
    logit += pb_ref[0, 0]
    o_ref[...] = (1.0 / (1.0 + jnp.exp(-logit)))[:, 0]


def _predict(h3, g, pwa, pwb, pb, block_m=BATCH):
    grid = (BATCH // block_m,)
    full = lambda m: (0, 0)
    return pl.pallas_call(
        _predict_body,
        grid=grid,
        in_specs=[
            pl.BlockSpec((block_m, D_GMF), lambda m: (m, 0)),
            pl.BlockSpec((block_m, D_GMF), lambda m: (m, 0)),
            pl.BlockSpec((64, 1), full),
            pl.BlockSpec((64, 1), full),
            pl.BlockSpec((1, 1), full),
        ],
        out_specs=pl.BlockSpec((block_m,), lambda m: (m,)),
        out_shape=jax.ShapeDtypeStruct((BATCH,), jnp.float32),
        compiler_params=pltpu.CompilerParams(
            dimension_semantics=("arbitrary",),
        ),
    )(h3, g, pwa, pwb, pb)


def kernel(users, items, user_emb_mlp, item_emb_mlp, user_emb_gmf,
           item_emb_gmf, mlp_w0, mlp_b0, mlp_w1, mlp_b1, mlp_w2, mlp_b2,
           pred_w, pred_b):
    users = users.astype(jnp.int32)
    items = items.astype(jnp.int32)

    u_rows, i_rows = _sc_gather_mlp(users, items, user_emb_mlp, item_emb_mlp)
    gmf_cat = _transpose_concat(user_emb_gmf.T, item_emb_gmf.T)
    (g,) = _sc_gather_gmf(users, items, gmf_cat)

    w0a = mlp_w0[:D_MLP]
    w0b = mlp_w0[D_MLP:]
    pwa = pred_w[:D_GMF]
    pwb = pred_w[D_GMF:]
    h3 = _mlp(u_rows, i_rows, w0a, w0b, mlp_b0.reshape(1, -1),
              mlp_w1, mlp_b1.reshape(1, -1), mlp_w2, mlp_b2.reshape(1, -1))
    out = _predict(h3, g, pwa, pwb, pred_b.reshape(1, 1))
    return out


# speedup vs baseline: 2.0450x; 1.0258x over previous
import functools

import jax
import jax.numpy as jnp
from jax import lax
from jax.experimental import pallas as pl
from jax.experimental.pallas import tpu as pltpu
from jax.experimental.pallas import tpu_sc as plsc

BATCH = 16384
D_MLP = 256
D_GMF = 64
N_ROWS = 100000

_NC = 2
_NS = 16
_NW = _NC * _NS
_BPW = BATCH // _NW
_CHUNK = 64
_NCHUNK = _BPW // _CHUNK

_sc_mesh = plsc.VectorSubcoreMesh(core_axis_name="c", subcore_axis_name="s")


@functools.partial(
    pl.kernel,
    mesh=_sc_mesh,
    out_type=[
        jax.ShapeDtypeStruct((BATCH, D_MLP), jnp.float32),
        jax.ShapeDtypeStruct((BATCH, D_MLP), jnp.float32),
    ],
    scratch_types=[
        pltpu.VMEM((_BPW,), jnp.int32),
        pltpu.VMEM((_BPW,), jnp.int32),
        pltpu.VMEM((2, _CHUNK, D_MLP), jnp.float32),
        pltpu.VMEM((2, _CHUNK, D_MLP), jnp.float32),
        pltpu.SemaphoreType.DMA,
        pltpu.SemaphoreType.DMA,
    ],
)
def _sc_gather_mlp(users_hbm, items_hbm, uemb_hbm, iemb_hbm,
                   out_u, out_i,
                   uidx_v, iidx_v, urows_v, irows_v, sem0, sem1):
    wid = lax.axis_index("s") * _NC + lax.axis_index("c")
    base = wid * _BPW
    sems = (sem0, sem1)

    pltpu.sync_copy(users_hbm.at[pl.ds(base, _BPW)], uidx_v)
    pltpu.sync_copy(items_hbm.at[pl.ds(base, _BPW)], iidx_v)

    def fire(k):
        p = k % 2
        uix = uidx_v.at[pl.ds(k * _CHUNK, _CHUNK)]
        iix = iidx_v.at[pl.ds(k * _CHUNK, _CHUNK)]
        return (
            pltpu.async_copy(uemb_hbm.at[uix], urows_v.at[p], sems[p]),
            pltpu.async_copy(iemb_hbm.at[iix], irows_v.at[p], sems[p]),
        )

    inflight = fire(0)
    for k in range(_NCHUNK):
        nxt = fire(k + 1) if k + 1 < _NCHUNK else None
        for c in inflight:
            c.wait()
        p = k % 2
        off = base + k * _CHUNK
        pltpu.sync_copy(urows_v.at[p], out_u.at[pl.ds(off, _CHUNK)])
        pltpu.sync_copy(irows_v.at[p], out_i.at[pl.ds(off, _CHUNK)])
        inflight = nxt


@functools.partial(
    pl.kernel,
    mesh=_sc_mesh,
    out_type=[
        jax.ShapeDtypeStruct((BATCH, D_GMF), jnp.float32),
    ],
    scratch_types=[
        pltpu.VMEM((_BPW,), jnp.int32),
        pltpu.VMEM((_BPW,), jnp.int32),
        pltpu.VMEM((2, _CHUNK, 2 * D_GMF), jnp.float32),
        pltpu.VMEM((2, _CHUNK, 2 * D_GMF), jnp.float32),
        pltpu.VMEM((_CHUNK, D_GMF), jnp.float32),
        pltpu.SemaphoreType.DMA,
        pltpu.SemaphoreType.DMA,
    ],
)
def _sc_gather_gmf(users_hbm, items_hbm, gcat_hbm, out_g,
                   uidx_v, iidx_v, ucat_v, icat_v, g_v, sem0, sem1):
    wid = lax.axis_index("s") * _NC + lax.axis_index("c")
    base = wid * _BPW
    sems = (sem0, sem1)

    pltpu.sync_copy(users_hbm.at[pl.ds(base, _BPW)], uidx_v)
    pltpu.sync_copy(items_hbm.at[pl.ds(base, _BPW)], iidx_v)

    def fire(k):
        p = k % 2
        uix = uidx_v.at[pl.ds(k * _CHUNK, _CHUNK)]
        iix = iidx_v.at[pl.ds(k * _CHUNK, _CHUNK)]
        return (
            pltpu.async_copy(gcat_hbm.at[uix], ucat_v.at[p], sems[p]),
            pltpu.async_copy(gcat_hbm.at[iix], icat_v.at[p], sems[p]),
        )

    inflight = fire(0)
    for k in range(_NCHUNK):
        nxt = fire(k + 1) if k + 1 < _NCHUNK else None
        for c in inflight:
            c.wait()
        p = k % 2
        off = base + k * _CHUNK

        def mul_body(r, mc):
            for c in range(D_GMF // 16):
                s = pl.ds(c * 16, 16)
                s_hi = pl.ds(D_GMF + c * 16, 16)
                g_v[r, s] = ucat_v[p, r, s] * icat_v[p, r, s_hi]
            return mc

        lax.fori_loop(0, _CHUNK, mul_body, 0)
        pltpu.sync_copy(g_v, out_g.at[pl.ds(off, _CHUNK)])
        inflight = nxt


def _tc_body(ut_ref, it_ref, o_ref):
    eye = jnp.eye(D_GMF, dtype=jnp.float32)
    zero = jnp.zeros((D_GMF, D_GMF), dtype=jnp.float32)
    e_top = jnp.concatenate([eye, zero], axis=1)
    e_bot = jnp.concatenate([zero, eye], axis=1)
    dn = (((0,), (0,)), ((), ()))
    o_ref[...] = (
        lax.dot_general(ut_ref[...], e_top, dn,
                        preferred_element_type=jnp.float32)
        + lax.dot_general(it_ref[...], e_bot, dn,
                          preferred_element_type=jnp.float32))


def _transpose_concat(ut, it, block_n=12800):
    grid = (pl.cdiv(N_ROWS, block_n),)
    return pl.pallas_call(
        _tc_body,
        grid=grid,
        in_specs=[
            pl.BlockSpec((D_GMF, block_n), lambda m: (0, m)),
            pl.BlockSpec((D_GMF, block_n), lambda m: (0, m)),
        ],
        out_specs=pl.BlockSpec((block_n, 2 * D_GMF), lambda m: (m, 0)),
        out_shape=jax.ShapeDtypeStruct((N_ROWS, 2 * D_GMF), jnp.float32),
        compiler_params=pltpu.CompilerParams(
            dimension_semantics=("arbitrary",),
        ),
    )(ut, it)


def _mlp_body(u_ref, i_ref, w0a_ref, w0b_ref, b0_ref, w1_ref,
              b1_ref, w2_ref, b2_ref, o_ref):
    h = jnp.dot(u_ref[...], w0a_ref[...], preferred_element_type=jnp.float32)
    h += jnp.dot(i_ref[...], w0b_ref[...], preferred_element_type=jnp.float32)
    h = jnp.maximum(h + b0_ref[...], 0.0)
    h = jnp.dot(h, w1_ref[...], preferred_element_type=jnp.float32)
    h = jnp.maximum(h + b1_ref[...], 0.0)
    h = jnp.dot(h, w2_ref[...], preferred_element_type=jnp.float32)
    o_ref[...] = jnp.maximum(h + b2_ref[...], 0.0)


def _mlp(u_rows, i_rows, w0a, w0b, b0, w1, b1, w2, b2, block_m=2048):
    grid = (BATCH // block_m,)
    full = lambda m: (0, 0)
    return pl.pallas_call(
        _mlp_body,
        grid=grid,
        in_specs=[
            pl.BlockSpec((block_m, D_MLP), lambda m: (m, 0)),
            pl.BlockSpec((block_m, D_MLP), lambda m: (m, 0)),
            pl.BlockSpec((D_MLP, 256), full),
            pl.BlockSpec((D_MLP, 256), full),
            pl.BlockSpec((1, 256), full),
            pl.BlockSpec((256, 128), full),
            pl.BlockSpec((1, 128), full),
            pl.BlockSpec((128, 64), full),
            pl.BlockSpec((1, 64), full),
        ],
        out_specs=pl.BlockSpec((block_m, D_GMF), lambda m: (m, 0)),
        out_shape=jax.ShapeDtypeStruct((BATCH, D_GMF), jnp.float32),
        compiler_params=pltpu.CompilerParams(
            dimension_semantics=("arbitrary",),
        ),
    )(u_rows, i_rows, w0a, w0b, b0, w1, b1, w2, b2)


def _predict_body(h_ref, g_ref, pwa_ref, pwb_ref, pb_ref, o_ref):
    logit = jnp.dot(h_ref[...], pwa_ref[...],
                    preferred_element_type=jnp.float32)
    logit += jnp.dot(g_ref[...], pwb_ref[...],
                     preferred_element_type=jnp.float32)
    lo = logit[:, 0] + pb_ref[0, 0]
    o_ref[...] = 1.0 / (1.0 + jnp.exp(-lo))


def _predict(h3, g, pwa, pwb, pb, block_m=BATCH):
    grid = (BATCH // block_m,)
    full = lambda m: (0, 0)
    return pl.pallas_call(
        _predict_body,
        grid=grid,
        in_specs=[
            pl.BlockSpec((block_m, D_GMF), lambda m: (m, 0)),
            pl.BlockSpec((block_m, D_GMF), lambda m: (m, 0)),
            pl.BlockSpec((64, 1), full),
            pl.BlockSpec((64, 1), full),
            pl.BlockSpec((1, 1), full),
        ],
        out_specs=pl.BlockSpec((block_m,), lambda m: (m,)),
        out_shape=jax.ShapeDtypeStruct((BATCH,), jnp.float32),
        compiler_params=pltpu.CompilerParams(
            dimension_semantics=("arbitrary",),
        ),
    )(h3, g, pwa, pwb, pb)


def kernel(users, items, user_emb_mlp, item_emb_mlp, user_emb_gmf,
           item_emb_gmf, mlp_w0, mlp_b0, mlp_w1, mlp_b1, mlp_w2, mlp_b2,
           pred_w, pred_b):
    users = users.astype(jnp.int32)
    items = items.astype(jnp.int32)

    u_rows, i_rows = _sc_gather_mlp(users, items, user_emb_mlp, item_emb_mlp)
    gmf_cat = _transpose_concat(user_emb_gmf.T, item_emb_gmf.T)
    (g,) = _sc_gather_gmf(users, items, gmf_cat)

    w0a = mlp_w0[:D_MLP]
    w0b = mlp_w0[D_MLP:]
    pwa = pred_w[:D_GMF]
    pwb = pred_w[D_GMF:]
    h3 = _mlp(u_rows, i_rows, w0a, w0b, mlp_b0.reshape(1, -1),
              mlp_w1, mlp_b1.reshape(1, -1), mlp_w2, mlp_b2.reshape(1, -1))
    out = _predict(h3, g, pwa, pwb, pred_b.reshape(1, 1))
    return out


# speedup vs baseline: 2.1146x; 1.0340x over previous
import functools

import jax
import jax.numpy as jnp
from jax import lax
from jax.experimental import pallas as pl
from jax.experimental.pallas import tpu as pltpu
from jax.experimental.pallas import tpu_sc as plsc

BATCH = 16384
D_MLP = 256
D_GMF = 64
N_ROWS = 100000

_NC = 2
_NS = 16
_NW = _NC * _NS
_BPW = BATCH // _NW
_CHUNK = 64
_NCHUNK = _BPW // _CHUNK

_sc_mesh = plsc.VectorSubcoreMesh(core_axis_name="c", subcore_axis_name="s")


@functools.partial(
    pl.kernel,
    mesh=_sc_mesh,
    out_type=[
        jax.ShapeDtypeStruct((BATCH, D_MLP), jnp.float32),
        jax.ShapeDtypeStruct((BATCH, D_MLP), jnp.float32),
    ],
    scratch_types=[
        pltpu.VMEM((_BPW,), jnp.int32),
        pltpu.VMEM((_BPW,), jnp.int32),
        pltpu.VMEM((2, _CHUNK, D_MLP), jnp.float32),
        pltpu.VMEM((2, _CHUNK, D_MLP), jnp.float32),
        pltpu.SemaphoreType.DMA,
        pltpu.SemaphoreType.DMA,
    ],
)
def _sc_gather_mlp(users_hbm, items_hbm, uemb_hbm, iemb_hbm,
                   out_u, out_i,
                   uidx_v, iidx_v, urows_v, irows_v, sem0, sem1):
    wid = lax.axis_index("s") * _NC + lax.axis_index("c")
    base = wid * _BPW
    sems = (sem0, sem1)

    pltpu.sync_copy(users_hbm.at[pl.ds(base, _BPW)], uidx_v)
    pltpu.sync_copy(items_hbm.at[pl.ds(base, _BPW)], iidx_v)

    def fire(k):
        p = k % 2
        uix = uidx_v.at[pl.ds(k * _CHUNK, _CHUNK)]
        iix = iidx_v.at[pl.ds(k * _CHUNK, _CHUNK)]
        return (
            pltpu.async_copy(uemb_hbm.at[uix], urows_v.at[p], sems[p]),
            pltpu.async_copy(iemb_hbm.at[iix], irows_v.at[p], sems[p]),
        )

    inflight = fire(0)
    for k in range(_NCHUNK):
        nxt = fire(k + 1) if k + 1 < _NCHUNK else None
        for c in inflight:
            c.wait()
        p = k % 2
        off = base + k * _CHUNK
        pltpu.sync_copy(urows_v.at[p], out_u.at[pl.ds(off, _CHUNK)])
        pltpu.sync_copy(irows_v.at[p], out_i.at[pl.ds(off, _CHUNK)])
        inflight = nxt


@functools.partial(
    pl.kernel,
    mesh=_sc_mesh,
    out_type=[
        jax.ShapeDtypeStruct((BATCH, D_GMF), jnp.float32),
    ],
    scratch_types=[
        pltpu.VMEM((_BPW,), jnp.int32),
        pltpu.VMEM((_BPW,), jnp.int32),
        pltpu.VMEM((2, _CHUNK, 2 * D_GMF), jnp.float32),
        pltpu.VMEM((2, _CHUNK, 2 * D_GMF), jnp.float32),
        pltpu.VMEM((_CHUNK, D_GMF), jnp.float32),
        pltpu.SemaphoreType.DMA,
        pltpu.SemaphoreType.DMA,
    ],
)
def _sc_gather_gmf(users_hbm, items_hbm, gcat_hbm, out_g,
                   uidx_v, iidx_v, ucat_v, icat_v, g_v, sem0, sem1):
    wid = lax.axis_index("s") * _NC + lax.axis_index("c")
    base = wid * _BPW
    sems = (sem0, sem1)

    pltpu.sync_copy(users_hbm.at[pl.ds(base, _BPW)], uidx_v)
    pltpu.sync_copy(items_hbm.at[pl.ds(base, _BPW)], iidx_v)

    def fire(k):
        p = k % 2
        uix = uidx_v.at[pl.ds(k * _CHUNK, _CHUNK)]
        iix = iidx_v.at[pl.ds(k * _CHUNK, _CHUNK)]
        return (
            pltpu.async_copy(gcat_hbm.at[uix], ucat_v.at[p], sems[p]),
            pltpu.async_copy(gcat_hbm.at[iix], icat_v.at[p], sems[p]),
        )

    inflight = fire(0)
    for k in range(_NCHUNK):
        nxt = fire(k + 1) if k + 1 < _NCHUNK else None
        for c in inflight:
            c.wait()
        p = k % 2
        off = base + k * _CHUNK

        def mul_body(r, mc):
            for c in range(D_GMF // 16):
                s = pl.ds(c * 16, 16)
                s_hi = pl.ds(D_GMF + c * 16, 16)
                g_v[r, s] = ucat_v[p, r, s] * icat_v[p, r, s_hi]
            return mc

        lax.fori_loop(0, _CHUNK, mul_body, 0)
        pltpu.sync_copy(g_v, out_g.at[pl.ds(off, _CHUNK)])
        inflight = nxt


def _tc_body(ut_ref, it_ref, o_ref):
    eye = jnp.eye(D_GMF, dtype=jnp.float32)
    zero = jnp.zeros((D_GMF, D_GMF), dtype=jnp.float32)
    e_top = jnp.concatenate([eye, zero], axis=1)
    e_bot = jnp.concatenate([zero, eye], axis=1)
    dn = (((0,), (0,)), ((), ()))
    o_ref[...] = (
        lax.dot_general(ut_ref[...], e_top, dn,
                        preferred_element_type=jnp.float32)
        + lax.dot_general(it_ref[...], e_bot, dn,
                          preferred_element_type=jnp.float32))


def _transpose_concat(ut, it, block_n=12800):
    grid = (pl.cdiv(N_ROWS, block_n),)
    return pl.pallas_call(
        _tc_body,
        grid=grid,
        in_specs=[
            pl.BlockSpec((D_GMF, block_n), lambda m: (0, m)),
            pl.BlockSpec((D_GMF, block_n), lambda m: (0, m)),
        ],
        out_specs=pl.BlockSpec((block_n, 2 * D_GMF), lambda m: (m, 0)),
        out_shape=jax.ShapeDtypeStruct((N_ROWS, 2 * D_GMF), jnp.float32),
        compiler_params=pltpu.CompilerParams(
            dimension_semantics=("arbitrary",),
        ),
    )(ut, it)


def _mlp_body(u_ref, i_ref, w0_ref, b0_ref, w1_ref,
              b1_ref, w2_ref, b2_ref, pw_ref, o_ref):
    h = jnp.dot(u_ref[...], w0_ref[0:D_MLP, :],
                preferred_element_type=jnp.float32)
    h += jnp.dot(i_ref[...], w0_ref[D_MLP:2 * D_MLP, :],
                 preferred_element_type=jnp.float32)
    h = jnp.maximum(h + b0_ref[...], 0.0)
    h = jnp.dot(h, w1_ref[...], preferred_element_type=jnp.float32)
    h = jnp.maximum(h + b1_ref[...], 0.0)
    h = jnp.dot(h, w2_ref[...], preferred_element_type=jnp.float32)
    h = jnp.maximum(h + b2_ref[...], 0.0)
    logit = jnp.dot(h, pw_ref[0:D_GMF, :],
                    preferred_element_type=jnp.float32)
    o_ref[...] = logit[:, 0]


def _mlp(u_rows, i_rows, w0, b0, w1, b1, w2, b2, pw, block_m=2048):
    grid = (BATCH // block_m,)
    full = lambda m: (0, 0)
    return pl.pallas_call(
        _mlp_body,
        grid=grid,
        in_specs=[
            pl.BlockSpec((block_m, D_MLP), lambda m: (m, 0)),
            pl.BlockSpec((block_m, D_MLP), lambda m: (m, 0)),
            pl.BlockSpec((2 * D_MLP, 256), full),
            pl.BlockSpec((1, 256), full),
            pl.BlockSpec((256, 128), full),
            pl.BlockSpec((1, 128), full),
            pl.BlockSpec((128, 64), full),
            pl.BlockSpec((1, 64), full),
            pl.BlockSpec((2 * D_GMF, 1), full),
        ],
        out_specs=pl.BlockSpec((block_m,), lambda m: (m,)),
        out_shape=jax.ShapeDtypeStruct((BATCH,), jnp.float32),
        compiler_params=pltpu.CompilerParams(
            dimension_semantics=("arbitrary",),
        ),
    )(u_rows, i_rows, w0, b0, w1, b1, w2, b2, pw)


def _predict_body(la_ref, g_ref, pw_ref, pb_ref, o_ref):
    logit = jnp.dot(g_ref[...], pw_ref[D_GMF:2 * D_GMF, :],
                    preferred_element_type=jnp.float32)
    lo = la_ref[...] + logit[:, 0] + pb_ref[0, 0]
    o_ref[...] = 1.0 / (1.0 + jnp.exp(-lo))


def _predict(logit_a, g, pw, pb, block_m=4096):
    grid = (BATCH // block_m,)
    full = lambda m: (0, 0)
    return pl.pallas_call(
        _predict_body,
        grid=grid,
        in_specs=[
            pl.BlockSpec((block_m,), lambda m: (m,)),
            pl.BlockSpec((block_m, D_GMF), lambda m: (m, 0)),
            pl.BlockSpec((2 * D_GMF, 1), full),
            pl.BlockSpec((1, 1), full),
        ],
        out_specs=pl.BlockSpec((block_m,), lambda m: (m,)),
        out_shape=jax.ShapeDtypeStruct((BATCH,), jnp.float32),
        compiler_params=pltpu.CompilerParams(
            dimension_semantics=("arbitrary",),
        ),
    )(logit_a, g, pw, pb)


def kernel(users, items, user_emb_mlp, item_emb_mlp, user_emb_gmf,
           item_emb_gmf, mlp_w0, mlp_b0, mlp_w1, mlp_b1, mlp_w2, mlp_b2,
           pred_w, pred_b):
    users = users.astype(jnp.int32)
    items = items.astype(jnp.int32)

    u_rows, i_rows = _sc_gather_mlp(users, items, user_emb_mlp, item_emb_mlp)
    gmf_cat = _transpose_concat(user_emb_gmf.T, item_emb_gmf.T)
    (g,) = _sc_gather_gmf(users, items, gmf_cat)

    logit_a = _mlp(u_rows, i_rows, mlp_w0, mlp_b0.reshape(1, -1),
                   mlp_w1, mlp_b1.reshape(1, -1), mlp_w2,
                   mlp_b2.reshape(1, -1), pred_w)
    out = _predict(logit_a, g, pred_w, pred_b.reshape(1, 1))
    return out
